# baseline (speedup 1.0000x reference)
import functools

import jax
import jax.numpy as jnp
from jax import lax
from jax.experimental import pallas as pl
from jax.experimental.pallas import tpu as pltpu
from jax.experimental.pallas import tpu_sc as plsc

_LANES = 16


def _rsqrt_newton(x):
    i = lax.bitcast_convert_type(x, jnp.int32)
    i = jnp.int32(0x5F3759DF) - lax.shift_right_logical(i, 1)
    y = lax.bitcast_convert_type(i, jnp.float32)
    for _ in range(3):
        y = y * (1.5 - 0.5 * x * y * y)
    return y


def _make_gather_norm(L, B, C, HW, NP):
    ROWS = L * B * NP
    NW = 32
    RPW = ROWS // NW
    CHUNKS = C // _LANES
    mesh = plsc.VectorSubcoreMesh(core_axis_name="c", subcore_axis_name="s")

    @functools.partial(
        pl.kernel,
        out_type=jax.ShapeDtypeStruct((ROWS * C,), jnp.float32),
        mesh=mesh,
        compiler_params=pltpu.CompilerParams(
            needs_layout_passes=False, disable_bounds_checks=True),
        scratch_types=[
            pltpu.VMEM((RPW,), jnp.int32),
            pltpu.VMEM((RPW * C,), jnp.int32),
            pltpu.VMEM((RPW * C,), jnp.float32),
            pltpu.SemaphoreType.DMA,
        ],
    )
    def gather_norm(feats_hbm, pids_hbm, out_hbm, pid_v, idx_v, rows_v, sem):
        wid = lax.axis_index("s") * 2 + lax.axis_index("c")
        r0 = wid * RPW
        g = r0 // NP
        n0 = r0 % NP
        lvl = g // B
        base = g * (C * HW)

        pltpu.sync_copy(pids_hbm.at[pl.ds(lvl * NP + n0, RPW)], pid_v)

        def build(j2, carry):
            iota = lax.broadcasted_iota(jnp.int32, (_LANES,), 0)
            for half in range(2):
                j = j2 * 2 + half
                p = plsc.load_gather(
                    pid_v, [jnp.full((_LANES,), j, jnp.int32)])
                cbase = iota * HW + base + p
                off = j * C

                def bchunk(k, cb, _off=off):
                    idx_v[pl.ds(_off + k * _LANES, _LANES)] = cb
                    return cb + (_LANES * HW)

                lax.fori_loop(0, CHUNKS, bchunk, cbase, unroll=False)
            return carry

        lax.fori_loop(0, RPW // 2, build, 0, unroll=False)

        pltpu.async_copy(feats_hbm.at[idx_v], rows_v, sem).wait()

        def norm(j2, carry):
            off0 = (j2 * 2) * C
            off1 = (j2 * 2 + 1) * C

            def achunk(k, accs):
                a0, a1 = accs
                v0 = rows_v[pl.ds(off0 + k * _LANES, _LANES)]
                v1 = rows_v[pl.ds(off1 + k * _LANES, _LANES)]
                return (a0 + v0 * v0, a1 + v1 * v1)

            a0, a1 = lax.fori_loop(
                0, CHUNKS, achunk,
                (jnp.zeros((_LANES,), jnp.float32),
                 jnp.zeros((_LANES,), jnp.float32)), unroll=False)
            rs = []
            for acc in (a0, a1):
                s = jnp.full((_LANES,), jnp.sum(acc))
                y = _rsqrt_newton(s)
                d = s * y + 1e-7
                y2 = _rsqrt_newton(d)
                rs.append(y2 * y2)

            def schunk(k, carry2):
                rows_v[pl.ds(off0 + k * _LANES, _LANES)] = (
                    rows_v[pl.ds(off0 + k * _LANES, _LANES)] * rs[0])
                rows_v[pl.ds(off1 + k * _LANES, _LANES)] = (
                    rows_v[pl.ds(off1 + k * _LANES, _LANES)] * rs[1])
                return carry2

            lax.fori_loop(0, CHUNKS, schunk, 0, unroll=False)
            return carry

        lax.fori_loop(0, RPW // 2, norm, 0, unroll=False)

        pltpu.sync_copy(rows_v, out_hbm.at[pl.ds(r0 * C, RPW * C)])

    return gather_norm


def kernel(feats, num_patches, patch_ids):
    L, B, C, H, W = feats.shape
    NP = patch_ids.shape[1]
    HW = H * W
    feats_flat = feats.reshape(-1)
    pids_flat = patch_ids.astype(jnp.int32).reshape(-1)
    out = _make_gather_norm(L, B, C, HW, NP)(feats_flat, pids_flat)
    return out.reshape(L, B * NP, C), patch_ids


# speedup vs baseline: 1.1334x; 1.1334x over previous
import functools

import jax
import jax.numpy as jnp
from jax import lax
from jax.experimental import pallas as pl
from jax.experimental.pallas import tpu as pltpu
from jax.experimental.pallas import tpu_sc as plsc

_LANES = 16


def _rsqrt_newton(x):
    i = lax.bitcast_convert_type(x, jnp.int32)
    i = jnp.int32(0x5F3759DF) - lax.shift_right_logical(i, 1)
    y = lax.bitcast_convert_type(i, jnp.float32)
    for _ in range(3):
        y = y * (1.5 - 0.5 * x * y * y)
    return y


def _make_gather_norm(L, B, C, HW, NP):
    ROWS = L * B * NP
    NW = 32
    RPW = ROWS // NW
    CHUNKS = C // _LANES
    mesh = plsc.VectorSubcoreMesh(core_axis_name="c", subcore_axis_name="s")

    @functools.partial(
        pl.kernel,
        out_type=jax.ShapeDtypeStruct((ROWS * C,), jnp.float32),
        mesh=mesh,
        compiler_params=pltpu.CompilerParams(
            needs_layout_passes=False, disable_bounds_checks=True),
        scratch_types=[
            pltpu.VMEM((RPW,), jnp.int32),
            pltpu.VMEM((RPW * C,), jnp.int32),
            pltpu.VMEM((RPW * C,), jnp.float32),
            pltpu.SemaphoreType.DMA,
        ],
    )
    def gather_norm(feats_hbm, pids_hbm, out_hbm, pid_v, idx_v, rows_v, sem):
        wid = lax.axis_index("s") * 2 + lax.axis_index("c")
        r0 = wid * RPW
        g = r0 // NP
        n0 = r0 % NP
        lvl = g // B
        base = g * (C * HW)

        pltpu.sync_copy(pids_hbm.at[pl.ds(lvl * NP + n0, RPW)], pid_v)

        def build(j4, carry):
            iota = lax.broadcasted_iota(jnp.int32, (_LANES,), 0)
            for sub in range(4):
                j = j4 * 4 + sub
                p = plsc.load_gather(
                    pid_v, [jnp.full((_LANES,), j, jnp.int32)])
                cbase = iota * HW + base + p
                off = j * C
                for k in range(CHUNKS):
                    idx_v[pl.ds(off + k * _LANES, _LANES)] = (
                        cbase + (k * _LANES * HW))
            return carry

        lax.fori_loop(0, RPW // 4, build, 0, unroll=False)

        pltpu.async_copy(feats_hbm.at[idx_v], rows_v, sem).wait()

        def norm(j4, carry):
            offs = [(j4 * 4 + sub) * C for sub in range(4)]
            accs = [jnp.zeros((_LANES,), jnp.float32) for _ in range(4)]
            for k in range(CHUNKS):
                for sub in range(4):
                    v = rows_v[pl.ds(offs[sub] + k * _LANES, _LANES)]
                    accs[sub] = accs[sub] + v * v
            rs = []
            for sub in range(4):
                s = jnp.full((_LANES,), jnp.sum(accs[sub]))
                y = _rsqrt_newton(s)
                d = s * y + 1e-7
                y2 = _rsqrt_newton(d)
                rs.append(y2 * y2)
            for k in range(CHUNKS):
                for sub in range(4):
                    rows_v[pl.ds(offs[sub] + k * _LANES, _LANES)] = (
                        rows_v[pl.ds(offs[sub] + k * _LANES, _LANES)]
                        * rs[sub])
            return carry

        lax.fori_loop(0, RPW // 4, norm, 0, unroll=False)

        pltpu.sync_copy(rows_v, out_hbm.at[pl.ds(r0 * C, RPW * C)])

    return gather_norm


def kernel(feats, num_patches, patch_ids):
    L, B, C, H, W = feats.shape
    NP = patch_ids.shape[1]
    HW = H * W
    feats_flat = feats.reshape(-1)
    pids_flat = patch_ids.astype(jnp.int32).reshape(-1)
    out = _make_gather_norm(L, B, C, HW, NP)(feats_flat, pids_flat)
    return out.reshape(L, B * NP, C), patch_ids
